# baseline (speedup 1.0000x reference)
import functools

import jax
import jax.numpy as jnp
import numpy as np
from jax import lax
from jax.experimental import pallas as pl
from jax.experimental.pallas import tpu as pltpu
from jax.experimental.pallas import tpu_sc as plsc

NUM_EMB = 100000
D = 32
HW_SIZE = 320000
B = 16384
L = 50

NC = 2
NS = 16
NW = NC * NS

_MESH = plsc.VectorSubcoreMesh(core_axis_name="c", subcore_axis_name="s")
_PARAMS = pltpu.CompilerParams(use_tc_tiling_on_sc=False)

TBL_WORDS = NUM_EMB * D
A_PER_W = TBL_WORDS // NW
A_CH = 10000
A_NCH = A_PER_W // A_CH
HW_PER_T = HW_SIZE // NS


@functools.partial(
    pl.kernel,
    out_type=jax.ShapeDtypeStruct((TBL_WORDS,), jnp.float32),
    mesh=_MESH,
    scratch_types=[
        pltpu.VMEM_SHARED((HW_SIZE,), jnp.float32),
        [pltpu.VMEM((A_CH,), jnp.int32)] * 2,
        [pltpu.VMEM((A_CH,), jnp.float32)] * 2,
        [pltpu.SemaphoreType.DMA] * 2,
    ],
    compiler_params=_PARAMS,
)
def _build_table(hw_hbm, widx_hbm, tbl_hbm, hw_sh, idx_v, val_v, sem):
    s = lax.axis_index("s")
    wid = s * NC + lax.axis_index("c")
    for j in range(2):
        off = s * HW_PER_T + j * A_CH
        pltpu.sync_copy(hw_hbm.at[pl.ds(off, A_CH)], val_v[j])
        pltpu.sync_copy(val_v[j], hw_sh.at[pl.ds(off, A_CH)])
    plsc.subcore_barrier()

    base = wid * A_PER_W
    pltpu.sync_copy(widx_hbm.at[pl.ds(base, A_CH)], idx_v[0])
    cps = [pltpu.async_copy(hw_sh.at[idx_v[0]], val_v[0], sem[0])]
    for k in range(1, A_NCH + 1):
        if k < A_NCH:
            pltpu.sync_copy(widx_hbm.at[pl.ds(base + k * A_CH, A_CH)],
                            idx_v[k % 2])
            cps.append(pltpu.async_copy(hw_sh.at[idx_v[k % 2]],
                                        val_v[k % 2], sem[k % 2]))
        cps[k - 1].wait()
        pltpu.sync_copy(val_v[(k - 1) % 2],
                        tbl_hbm.at[pl.ds(base + (k - 1) * A_CH, A_CH)])


BAGS_PER_W = B // NW
BAGS_PER_SC = B // NC
G = 16
GL = G * L
B_NCH = BAGS_PER_W // G
NBUF = 4


@functools.partial(
    pl.kernel,
    out_type=jax.ShapeDtypeStruct((B, D), jnp.float32),
    mesh=_MESH,
    scratch_types=[
        pltpu.VMEM_SHARED((BAGS_PER_SC, D), jnp.float32),
        [pltpu.VMEM((GL,), jnp.int32)] * NBUF,
        [pltpu.VMEM((GL, D), jnp.float32)] * NBUF,
        [pltpu.VMEM((GL,), jnp.int32)] * NBUF,
        [pltpu.SemaphoreType.DMA] * NBUF,
        [pltpu.SemaphoreType.DMA] * NBUF,
    ],
    compiler_params=_PARAMS,
)
def _bag_sum(tbl_hbm, x_hbm, sidx_hbm, out_hbm, acc_sh, xidx_v, rows_v,
             sidx_v, gsem, ssem):
    s = lax.axis_index("s")
    wid = s * NC + lax.axis_index("c")
    bag0 = wid * BAGS_PER_W
    accrow0 = s * BAGS_PER_W

    def start(k):
        pltpu.sync_copy(x_hbm.at[pl.ds((bag0 + k * G) * L, GL)],
                        xidx_v[k % NBUF])
        return pltpu.async_copy(tbl_hbm.at[xidx_v[k % NBUF]],
                                rows_v[k % NBUF], gsem[k % NBUF])

    cps = [start(0), start(1), start(2)]

    def zbody(i, _):
        z = jnp.zeros((16,), jnp.float32)
        rows_v[3][i, pl.ds(0, 16)] = z
        rows_v[3][i, pl.ds(16, 16)] = z
        return 0
    lax.fori_loop(0, BAGS_PER_W, zbody, 0)
    pltpu.sync_copy(rows_v[3].at[pl.ds(0, BAGS_PER_W)],
                    acc_sh.at[pl.ds(accrow0, BAGS_PER_W)])

    scs = []
    for kk in range(B_NCH):
        k3 = kk + NBUF - 1
        if k3 < B_NCH:
            if k3 >= NBUF:
                scs[k3 - NBUF].wait()
            cps.append(start(k3))
        pltpu.sync_copy(sidx_hbm.at[pl.ds((s * B_NCH + kk) * GL, GL)],
                        sidx_v[kk % NBUF])
        cps[kk].wait()
        scs.append(pltpu.async_copy(rows_v[kk % NBUF],
                                    acc_sh.at[sidx_v[kk % NBUF]],
                                    ssem[kk % NBUF], add=True))

    for j in range(B_NCH - NBUF, B_NCH):
        scs[j].wait()
    pltpu.sync_copy(acc_sh.at[pl.ds(accrow0, BAGS_PER_W)],
                    rows_v[0].at[pl.ds(0, BAGS_PER_W)])
    pltpu.sync_copy(rows_v[0].at[pl.ds(0, BAGS_PER_W)],
                    out_hbm.at[pl.ds(bag0, BAGS_PER_W)])


_SIDX = jnp.asarray(
    (np.arange(NS, dtype=np.int32)[:, None, None] * BAGS_PER_W
     + np.arange(B_NCH, dtype=np.int32)[None, :, None] * G
     + np.arange(GL, dtype=np.int32)[None, None, :] // L)
    .reshape(-1))


def kernel(x, hashed_weight, weight_idx):
    tbl_flat = _build_table(hashed_weight, weight_idx.reshape(-1))
    tbl = tbl_flat.reshape(NUM_EMB, D)
    return _bag_sum(tbl, x.reshape(-1), _SIDX)
